# Initial kernel scaffold; baseline (speedup 1.0000x reference)
import jax
import jax.numpy as jnp
from jax.experimental import pallas as pl


def kernel(x, edge_index, substring_embed, batch, Wm0, bm0, Wu0, bu0, Wm1, bm1, Wu1, bu1, Wo, bo):
    raise NotImplementedError("write your pallas kernel here")


# speedup vs baseline: 4.5677x; 4.5677x over previous
import functools

import jax
import jax.numpy as jnp
from jax import lax
from jax.experimental import pallas as pl
from jax.experimental.pallas import tpu as pltpu
from jax.experimental.pallas import tpu_sc as plsc

NSC = 2
NSUB = 16
F32 = jnp.float32


def _dot_nt(a, b):
    return lax.dot_general(a, b, (((1,), (1,)), ((), ())),
                           preferred_element_type=F32)


def _largest_div(n, cap):
    for v in range(cap, 0, -1):
        if n % v == 0:
            return v
    return 1


def _make_sc_aggregate(n, h, nch, c, with_gather):
    rpt = n // NSUB

    mesh = plsc.VectorSubcoreMesh(core_axis_name="c", subcore_axis_name="s")
    out_type = jax.ShapeDtypeStruct((NSC, n, h), F32)
    scratch = [
        pltpu.VMEM((c,), jnp.int32),
        pltpu.VMEM((c,), jnp.int32),
        pltpu.VMEM((c, h), F32),
        pltpu.VMEM_SHARED((n, h), F32),
        pltpu.SemaphoreType.DMA,
    ]

    @functools.partial(pl.kernel, mesh=mesh, out_type=out_type,
                       scratch_types=scratch)
    def sc_kernel(xt_hbm, src_hbm, dst_hbm, p_hbm, sidx, didx, rows,
                  acc_sh, sem):
        cc = lax.axis_index("c")
        ss = lax.axis_index("s")
        tid = cc * NSUB + ss
        zeros16 = jnp.zeros((16,), F32)

        def zrow(j, carry):
            for l in range(h // 16):
                rows[j, pl.ds(l * 16, 16)] = zeros16
            return carry
        lax.fori_loop(0, c, zrow, 0)

        row0 = ss * rpt
        for k in range(rpt // c):
            pltpu.sync_copy(rows, acc_sh.at[pl.ds(row0 + k * c, c)])
        if not with_gather:
            ones16 = jnp.ones((16,), F32)

            def orow(j, carry):
                for l in range(h // 16):
                    rows[j, pl.ds(l * 16, 16)] = ones16
                return carry
            lax.fori_loop(0, c, orow, 0)
        plsc.subcore_barrier()

        def chunk(j, carry):
            if with_gather:
                pltpu.sync_copy(src_hbm.at[tid, j], sidx)
            pltpu.sync_copy(dst_hbm.at[tid, j], didx)
            if with_gather:
                pltpu.async_copy(xt_hbm.at[sidx], rows, sem).wait()
            pltpu.sync_copy(rows, acc_sh.at[didx], add=True)
            return carry
        lax.fori_loop(0, nch, chunk, 0)

        plsc.subcore_barrier()
        pltpu.sync_copy(acc_sh.at[pl.ds(row0, rpt)],
                        p_hbm.at[cc, pl.ds(row0, rpt)])

    return sc_kernel


def _onehot(b_blk, g):
    io = lax.broadcasted_iota(jnp.int32, (1, g), 1)
    return (b_blk == io).astype(F32)


def _k_msg0(x_ref, b_ref, se_ref, wm_ref, bm_ref, o_ref, *, d, g):
    wx = wm_ref[:, :d]
    wc = wm_ref[:, d:]
    cw = _dot_nt(se_ref[...], wc)
    oh = _onehot(b_ref[...], g)
    o_ref[...] = (_dot_nt(x_ref[...], wx)
                  + lax.dot_general(oh, cw, (((1,), (0,)), ((), ())),
                                    preferred_element_type=F32)
                  + bm_ref[...])


def _k_update0(x_ref, b_ref, se_ref, p_ref, c_ref, wu_ref, bu_ref,
               wm1_ref, bm1_ref, h1_ref, xt1_ref, *, d, cdim, g):
    pa = p_ref[0] + p_ref[1]
    cnt = c_ref[0, :, 0:1] + c_ref[1, :, 0:1]
    aggr = pa / jnp.maximum(cnt, 1.0)
    wua = wu_ref[:, :d]
    wub = wu_ref[:, d:d + cdim]
    wuc = wu_ref[:, d + cdim:]
    cw = _dot_nt(se_ref[...], wub)
    oh = _onehot(b_ref[...], g)
    h1 = (_dot_nt(x_ref[...], wua)
          + lax.dot_general(oh, cw, (((1,), (0,)), ((), ())),
                            preferred_element_type=F32)
          + _dot_nt(aggr, wuc) + bu_ref[...])
    h1 = jnp.maximum(h1, 0.0)
    h1_ref[...] = h1
    xt1_ref[...] = _dot_nt(h1, wm1_ref[...]) + bm1_ref[...]


def _k_update1(h1_ref, p_ref, c_ref, wu_ref, bu_ref, wo_ref, bo_ref,
               o_ref, *, hdim):
    pa = p_ref[0] + p_ref[1]
    cnt = c_ref[0, :, 0:1] + c_ref[1, :, 0:1]
    aggr = pa / jnp.maximum(cnt, 1.0)
    wa = wu_ref[:, :hdim]
    wb = wu_ref[:, hdim:]
    h2 = _dot_nt(h1_ref[...], wa) + _dot_nt(aggr, wb) + bu_ref[...]
    h2 = jnp.maximum(h2, 0.0)
    o_ref[...] = jnp.sum(h2 * wo_ref[...], axis=1, keepdims=True) + bo_ref[0, 0]


def kernel(x, edge_index, substring_embed, batch, Wm0, bm0, Wu0, bu0,
           Wm1, bm1, Wu1, bu1, Wo, bo):
    n, d = x.shape
    g, cdim = substring_embed.shape
    h = Wm0.shape[0]
    e = edge_index.shape[1]

    ntile = NSC * NSUB
    ept = e // ntile
    ch = _largest_div(ept, 80)
    nch = ept // ch
    npad = -(-n // (8 * NSUB)) * (8 * NSUB)

    bn = _largest_div(n, 2000)
    grid = n // bn

    src3 = edge_index[0].reshape(ntile, nch, ch)
    dst3 = edge_index[1].reshape(ntile, nch, ch)
    b2d = batch.reshape(n, 1)
    bm0r = bm0.reshape(1, h)
    bu0r = bu0.reshape(1, h)
    bm1r = bm1.reshape(1, h)
    bu1r = bu1.reshape(1, h)
    bor = bo.reshape(1, 1)

    full = lambda s: pl.BlockSpec(s, lambda i: tuple(0 for _ in s))
    rowblk = lambda w: pl.BlockSpec((bn, w), lambda i: (i, 0))
    pblk = lambda w: pl.BlockSpec((NSC, bn, w), lambda i: (0, i, 0))

    xt0 = pl.pallas_call(
        functools.partial(_k_msg0, d=d, g=g),
        grid=(grid,),
        in_specs=[rowblk(d), rowblk(1), full((g, cdim)),
                  full((h, d + cdim)), full((1, h))],
        out_specs=rowblk(h),
        out_shape=jax.ShapeDtypeStruct((n, h), F32),
    )(x, b2d, substring_embed, Wm0, bm0r)

    sccnt = _make_sc_aggregate(npad, h, nch, ch, False)
    cnt = sccnt(x, src3, dst3)
    scagg = _make_sc_aggregate(npad, h, nch, ch, True)
    p0 = scagg(xt0, src3, dst3)

    h1, xt1 = pl.pallas_call(
        functools.partial(_k_update0, d=d, cdim=cdim, g=g),
        grid=(grid,),
        in_specs=[rowblk(d), rowblk(1), full((g, cdim)), pblk(h), pblk(h),
                  full((h, d + cdim + h)), full((1, h)),
                  full((h, h)), full((1, h))],
        out_specs=[rowblk(h), rowblk(h)],
        out_shape=[jax.ShapeDtypeStruct((n, h), F32),
                   jax.ShapeDtypeStruct((n, h), F32)],
    )(x, b2d, substring_embed, p0, cnt, Wu0, bu0r, Wm1, bm1r)

    p1 = scagg(xt1, src3, dst3)

    out2d = pl.pallas_call(
        functools.partial(_k_update1, hdim=h),
        grid=(grid,),
        in_specs=[rowblk(h), pblk(h), pblk(h),
                  full((h, 2 * h)), full((1, h)),
                  full((1, h)), full((1, 1))],
        out_specs=rowblk(1),
        out_shape=jax.ShapeDtypeStruct((n, 1), F32),
    )(h1, p1, cnt, Wu1, bu1r, Wo, bor)

    return out2d.reshape(n)


# speedup vs baseline: 5.5893x; 1.2237x over previous
import functools

import jax
import jax.numpy as jnp
from jax import lax
from jax.experimental import pallas as pl
from jax.experimental.pallas import tpu as pltpu
from jax.experimental.pallas import tpu_sc as plsc

NSC = 2
NSUB = 16
F32 = jnp.float32


def _dot_nt(a, b):
    return lax.dot_general(a, b, (((1,), (1,)), ((), ())),
                           preferred_element_type=F32)


def _largest_div(n, cap):
    for v in range(cap, 0, -1):
        if n % v == 0:
            return v
    return 1


def _make_sc_aggregate(n, h, nch, c, with_gather):
    rpt = n // NSUB

    mesh = plsc.VectorSubcoreMesh(core_axis_name="c", subcore_axis_name="s")
    out_type = jax.ShapeDtypeStruct((NSC, n, h), F32)
    scratch = [
        pltpu.VMEM((2, c), jnp.int32),
        pltpu.VMEM((2, c), jnp.int32),
        pltpu.VMEM((2, c, h), F32),
        pltpu.VMEM_SHARED((n, h), F32),
        pltpu.SemaphoreType.DMA,
        pltpu.SemaphoreType.DMA,
        pltpu.SemaphoreType.DMA,
        pltpu.SemaphoreType.DMA,
    ]

    @functools.partial(pl.kernel, mesh=mesh, out_type=out_type,
                       scratch_types=scratch)
    def sc_kernel(xt_hbm, src_hbm, dst_hbm, p_hbm, sidx, didx, rows,
                  acc_sh, g0, g1, s0, s1):
        cc = lax.axis_index("c")
        ss = lax.axis_index("s")
        tid = cc * NSUB + ss
        gsem = (g0, g1)
        ssem = (s0, s1)
        zeros16 = jnp.zeros((16,), F32)

        def zrow(j, carry):
            for l in range(h // 16):
                rows[0, j, pl.ds(l * 16, 16)] = zeros16
                rows[1, j, pl.ds(l * 16, 16)] = zeros16
            return carry
        lax.fori_loop(0, c, zrow, 0)

        row0 = ss * rpt
        for k in range(rpt // c):
            pltpu.sync_copy(rows.at[0], acc_sh.at[pl.ds(row0 + k * c, c)])
        if not with_gather:
            ones16 = jnp.ones((16,), F32)

            def orow(j, carry):
                for l in range(h // 16):
                    rows[0, j, pl.ds(l * 16, 16)] = ones16
                    rows[1, j, pl.ds(l * 16, 16)] = ones16
                return carry
            lax.fori_loop(0, c, orow, 0)
        plsc.subcore_barrier()

        def idx_load(j, p):
            if with_gather:
                pltpu.sync_copy(src_hbm.at[tid, j], sidx.at[p])
            pltpu.sync_copy(dst_hbm.at[tid, j], didx.at[p])

        def gather_start(p):
            pltpu.async_copy(xt_hbm.at[sidx.at[p]], rows.at[p], gsem[p])

        def gather_wait(p):
            pltpu.make_async_copy(xt_hbm.at[sidx.at[p]], rows.at[p],
                                  gsem[p]).wait()

        def scatter_start(p):
            pltpu.async_copy(rows.at[p], acc_sh.at[didx.at[p]], ssem[p],
                             add=True)

        def scatter_wait(p):
            pltpu.make_async_copy(rows.at[p], acc_sh.at[didx.at[p]],
                                  ssem[p]).wait()

        def step(j, p):
            if with_gather:
                gather_wait(p)
            scatter_start(p)

            @pl.when(j > 0)
            def _():
                scatter_wait(1 - p)

            @pl.when(j + 1 < nch)
            def _():
                idx_load(j + 1, 1 - p)
                if with_gather:
                    gather_start(1 - p)

        idx_load(0, 0)
        if with_gather:
            gather_start(0)

        def pair(i, carry):
            step(2 * i, 0)
            step(2 * i + 1, 1)
            return carry
        lax.fori_loop(0, nch // 2, pair, 0)
        if nch % 2:
            step(nch - 1, 0)
            scatter_wait(0)
        else:
            scatter_wait(1)

        plsc.subcore_barrier()
        pltpu.sync_copy(acc_sh.at[pl.ds(row0, rpt)],
                        p_hbm.at[cc, pl.ds(row0, rpt)])

    return sc_kernel


def _onehot(b_blk, g):
    io = lax.broadcasted_iota(jnp.int32, (1, g), 1)
    return (b_blk == io).astype(F32)


def _k_msg0(x_ref, b_ref, se_ref, wm_ref, bm_ref, o_ref, *, d, g):
    wx = wm_ref[:, :d]
    wc = wm_ref[:, d:]
    cw = _dot_nt(se_ref[...], wc)
    oh = _onehot(b_ref[...], g)
    o_ref[...] = (_dot_nt(x_ref[...], wx)
                  + lax.dot_general(oh, cw, (((1,), (0,)), ((), ())),
                                    preferred_element_type=F32)
                  + bm_ref[...])


def _k_update0(x_ref, b_ref, se_ref, p_ref, c_ref, wu_ref, bu_ref,
               wm1_ref, bm1_ref, h1_ref, xt1_ref, *, d, cdim, g):
    pa = p_ref[0] + p_ref[1]
    cnt = c_ref[0, :, 0:1] + c_ref[1, :, 0:1]
    aggr = pa / jnp.maximum(cnt, 1.0)
    wua = wu_ref[:, :d]
    wub = wu_ref[:, d:d + cdim]
    wuc = wu_ref[:, d + cdim:]
    cw = _dot_nt(se_ref[...], wub)
    oh = _onehot(b_ref[...], g)
    h1 = (_dot_nt(x_ref[...], wua)
          + lax.dot_general(oh, cw, (((1,), (0,)), ((), ())),
                            preferred_element_type=F32)
          + _dot_nt(aggr, wuc) + bu_ref[...])
    h1 = jnp.maximum(h1, 0.0)
    h1_ref[...] = h1
    xt1_ref[...] = _dot_nt(h1, wm1_ref[...]) + bm1_ref[...]


def _k_update1(h1_ref, p_ref, c_ref, wu_ref, bu_ref, wo_ref, bo_ref,
               o_ref, *, hdim):
    pa = p_ref[0] + p_ref[1]
    cnt = c_ref[0, :, 0:1] + c_ref[1, :, 0:1]
    aggr = pa / jnp.maximum(cnt, 1.0)
    wa = wu_ref[:, :hdim]
    wb = wu_ref[:, hdim:]
    h2 = _dot_nt(h1_ref[...], wa) + _dot_nt(aggr, wb) + bu_ref[...]
    h2 = jnp.maximum(h2, 0.0)
    o_ref[...] = jnp.sum(h2 * wo_ref[...], axis=1, keepdims=True) + bo_ref[0, 0]


def kernel(x, edge_index, substring_embed, batch, Wm0, bm0, Wu0, bu0,
           Wm1, bm1, Wu1, bu1, Wo, bo):
    n, d = x.shape
    g, cdim = substring_embed.shape
    h = Wm0.shape[0]
    e = edge_index.shape[1]

    ntile = NSC * NSUB
    ept = e // ntile
    ch = _largest_div(ept, 80)
    nch = ept // ch
    npad = -(-n // (8 * NSUB)) * (8 * NSUB)

    bn = _largest_div(n, 2000)
    grid = n // bn

    src3 = edge_index[0].reshape(ntile, nch, ch)
    dst3 = edge_index[1].reshape(ntile, nch, ch)
    b2d = batch.reshape(n, 1)
    bm0r = bm0.reshape(1, h)
    bu0r = bu0.reshape(1, h)
    bm1r = bm1.reshape(1, h)
    bu1r = bu1.reshape(1, h)
    bor = bo.reshape(1, 1)

    full = lambda s: pl.BlockSpec(s, lambda i: tuple(0 for _ in s))
    rowblk = lambda w: pl.BlockSpec((bn, w), lambda i: (i, 0))
    pblk = lambda w: pl.BlockSpec((NSC, bn, w), lambda i: (0, i, 0))

    xt0 = pl.pallas_call(
        functools.partial(_k_msg0, d=d, g=g),
        grid=(grid,),
        in_specs=[rowblk(d), rowblk(1), full((g, cdim)),
                  full((h, d + cdim)), full((1, h))],
        out_specs=rowblk(h),
        out_shape=jax.ShapeDtypeStruct((n, h), F32),
    )(x, b2d, substring_embed, Wm0, bm0r)

    sccnt = _make_sc_aggregate(npad, h, nch, ch, False)
    cnt = sccnt(x, src3, dst3)
    scagg = _make_sc_aggregate(npad, h, nch, ch, True)
    p0 = scagg(xt0, src3, dst3)

    h1, xt1 = pl.pallas_call(
        functools.partial(_k_update0, d=d, cdim=cdim, g=g),
        grid=(grid,),
        in_specs=[rowblk(d), rowblk(1), full((g, cdim)), pblk(h), pblk(h),
                  full((h, d + cdim + h)), full((1, h)),
                  full((h, h)), full((1, h))],
        out_specs=[rowblk(h), rowblk(h)],
        out_shape=[jax.ShapeDtypeStruct((n, h), F32),
                   jax.ShapeDtypeStruct((n, h), F32)],
    )(x, b2d, substring_embed, p0, cnt, Wu0, bu0r, Wm1, bm1r)

    p1 = scagg(xt1, src3, dst3)

    out2d = pl.pallas_call(
        functools.partial(_k_update1, hdim=h),
        grid=(grid,),
        in_specs=[rowblk(h), pblk(h), pblk(h),
                  full((h, 2 * h)), full((1, h)),
                  full((1, h)), full((1, 1))],
        out_specs=rowblk(1),
        out_shape=jax.ShapeDtypeStruct((n, 1), F32),
    )(h1, p1, cnt, Wu1, bu1r, Wo, bor)

    return out2d.reshape(n)


# speedup vs baseline: 8.3665x; 1.4969x over previous
import functools

import jax
import jax.numpy as jnp
from jax import lax
from jax.experimental import pallas as pl
from jax.experimental.pallas import tpu as pltpu
from jax.experimental.pallas import tpu_sc as plsc

NSC = 2
NSUB = 16
F32 = jnp.float32


def _dot_nt(a, b):
    return lax.dot_general(a, b, (((1,), (1,)), ((), ())),
                           preferred_element_type=F32)


def _largest_div(n, cap):
    for v in range(cap, 0, -1):
        if n % v == 0:
            return v
    return 1


def _make_sc_aggregate(n, h, nch, c, with_gather):
    rpt = n // NSUB

    mesh = plsc.VectorSubcoreMesh(core_axis_name="c", subcore_axis_name="s")
    out_type = jax.ShapeDtypeStruct((NSC, n, h), F32)
    scratch = [
        pltpu.VMEM((2, c), jnp.int32),
        pltpu.VMEM((nch, c), jnp.int32),
        pltpu.VMEM((2, c, h), F32),
        pltpu.VMEM_SHARED((n, h), F32),
        pltpu.SemaphoreType.DMA,
        pltpu.SemaphoreType.DMA,
        pltpu.SemaphoreType.DMA,
        pltpu.SemaphoreType.DMA,
        pltpu.SemaphoreType.DMA,
        pltpu.SemaphoreType.DMA,
    ]

    @functools.partial(pl.kernel, mesh=mesh, out_type=out_type,
                       scratch_types=scratch)
    def sc_kernel(xt_hbm, src_hbm, dst_hbm, p_hbm, sidx, didx, rows,
                  acc_sh, g0, g1, s0, s1, i0, i1):
        cc = lax.axis_index("c")
        ss = lax.axis_index("s")
        tid = cc * NSUB + ss
        gsem = (g0, g1)
        ssem = (s0, s1)
        isem = (i0, i1)
        zeros16 = jnp.zeros((16,), F32)

        def zrow(j, carry):
            for l in range(h // 16):
                rows[0, j, pl.ds(l * 16, 16)] = zeros16
                rows[1, j, pl.ds(l * 16, 16)] = zeros16
            return carry
        lax.fori_loop(0, c, zrow, 0)

        row0 = ss * rpt
        for k in range(rpt // c):
            pltpu.sync_copy(rows.at[0], acc_sh.at[pl.ds(row0 + k * c, c)])
        if not with_gather:
            ones16 = jnp.ones((16,), F32)

            def orow(j, carry):
                for l in range(h // 16):
                    rows[0, j, pl.ds(l * 16, 16)] = ones16
                    rows[1, j, pl.ds(l * 16, 16)] = ones16
                return carry
            lax.fori_loop(0, c, orow, 0)
        pltpu.sync_copy(dst_hbm.at[tid], didx)
        plsc.subcore_barrier()

        def sidx_start(j, p):
            pltpu.async_copy(src_hbm.at[tid, j], sidx.at[p], isem[p])

        def sidx_wait(j, p):
            pltpu.make_async_copy(src_hbm.at[tid, j], sidx.at[p],
                                  isem[p]).wait()

        def gather_start(p):
            pltpu.async_copy(xt_hbm.at[sidx.at[p]], rows.at[p], gsem[p])

        def gather_wait(p):
            pltpu.make_async_copy(xt_hbm.at[sidx.at[p]], rows.at[p],
                                  gsem[p]).wait()

        def scatter_start(j, p):
            pltpu.async_copy(rows.at[p], acc_sh.at[didx.at[j]], ssem[p],
                             add=True)

        def scatter_wait(j, p):
            pltpu.make_async_copy(rows.at[p], acc_sh.at[didx.at[j]],
                                  ssem[p]).wait()

        def step(j, p):
            if with_gather:
                gather_wait(p)
            scatter_start(j, p)

            @pl.when(j > 0)
            def _():
                scatter_wait(j - 1, 1 - p)

            if with_gather:
                @pl.when(j + 1 < nch)
                def _():
                    sidx_wait(j + 1, 1 - p)
                    gather_start(1 - p)

                @pl.when(j + 2 < nch)
                def _():
                    sidx_start(j + 2, p)

        if with_gather:
            sidx_start(0, 0)
            sidx_wait(0, 0)
            gather_start(0)
            sidx_start(1, 1)

        def pair(i, carry):
            step(2 * i, 0)
            step(2 * i + 1, 1)
            return carry
        lax.fori_loop(0, nch // 2, pair, 0)
        if nch % 2:
            step(nch - 1, 0)
            scatter_wait(nch - 1, 0)
        else:
            scatter_wait(nch - 1, 1)

        plsc.subcore_barrier()
        pltpu.sync_copy(acc_sh.at[pl.ds(row0, rpt)],
                        p_hbm.at[cc, pl.ds(row0, rpt)])

    return sc_kernel


def _onehot(b_blk, g):
    io = lax.broadcasted_iota(jnp.int32, (1, g), 1)
    return (b_blk == io).astype(F32)


def _k_msg0(x_ref, b_ref, se_ref, wm_ref, bm_ref, o_ref, *, d, g):
    wx = wm_ref[:, :d]
    wc = wm_ref[:, d:]
    cw = _dot_nt(se_ref[...], wc)
    oh = _onehot(b_ref[...], g)
    o_ref[...] = (_dot_nt(x_ref[...], wx)
                  + lax.dot_general(oh, cw, (((1,), (0,)), ((), ())),
                                    preferred_element_type=F32)
                  + bm_ref[...])


def _k_update0(x_ref, b_ref, se_ref, p_ref, c_ref, wu_ref, bu_ref,
               wm1_ref, bm1_ref, h1_ref, xt1_ref, *, d, cdim, g):
    pa = p_ref[0] + p_ref[1]
    cnt = c_ref[0, :, 0:1] + c_ref[1, :, 0:1]
    aggr = pa / jnp.maximum(cnt, 1.0)
    wua = wu_ref[:, :d]
    wub = wu_ref[:, d:d + cdim]
    wuc = wu_ref[:, d + cdim:]
    cw = _dot_nt(se_ref[...], wub)
    oh = _onehot(b_ref[...], g)
    h1 = (_dot_nt(x_ref[...], wua)
          + lax.dot_general(oh, cw, (((1,), (0,)), ((), ())),
                            preferred_element_type=F32)
          + _dot_nt(aggr, wuc) + bu_ref[...])
    h1 = jnp.maximum(h1, 0.0)
    h1_ref[...] = h1
    xt1_ref[...] = _dot_nt(h1, wm1_ref[...]) + bm1_ref[...]


def _k_update1(h1_ref, p_ref, c_ref, wu_ref, bu_ref, wo_ref, bo_ref,
               o_ref, *, hdim):
    pa = p_ref[0] + p_ref[1]
    cnt = c_ref[0, :, 0:1] + c_ref[1, :, 0:1]
    aggr = pa / jnp.maximum(cnt, 1.0)
    wa = wu_ref[:, :hdim]
    wb = wu_ref[:, hdim:]
    h2 = _dot_nt(h1_ref[...], wa) + _dot_nt(aggr, wb) + bu_ref[...]
    h2 = jnp.maximum(h2, 0.0)
    o_ref[...] = jnp.sum(h2 * wo_ref[...], axis=1, keepdims=True) + bo_ref[0, 0]


def kernel(x, edge_index, substring_embed, batch, Wm0, bm0, Wu0, bu0,
           Wm1, bm1, Wu1, bu1, Wo, bo):
    n, d = x.shape
    g, cdim = substring_embed.shape
    h = Wm0.shape[0]
    e = edge_index.shape[1]

    ntile = NSC * NSUB
    ept = e // ntile
    ch = _largest_div(ept, 80)
    nch = ept // ch
    npad = -(-n // (8 * NSUB)) * (8 * NSUB)

    bn = _largest_div(n, 2000)
    grid = n // bn

    src3 = edge_index[0].reshape(ntile, nch, ch)
    dst3 = edge_index[1].reshape(ntile, nch, ch)
    b2d = batch.reshape(n, 1)
    bm0r = bm0.reshape(1, h)
    bu0r = bu0.reshape(1, h)
    bm1r = bm1.reshape(1, h)
    bu1r = bu1.reshape(1, h)
    bor = bo.reshape(1, 1)

    full = lambda s: pl.BlockSpec(s, lambda i: tuple(0 for _ in s))
    rowblk = lambda w: pl.BlockSpec((bn, w), lambda i: (i, 0))
    pblk = lambda w: pl.BlockSpec((NSC, bn, w), lambda i: (0, i, 0))

    xt0 = pl.pallas_call(
        functools.partial(_k_msg0, d=d, g=g),
        grid=(grid,),
        in_specs=[rowblk(d), rowblk(1), full((g, cdim)),
                  full((h, d + cdim)), full((1, h))],
        out_specs=rowblk(h),
        out_shape=jax.ShapeDtypeStruct((n, h), F32),
    )(x, b2d, substring_embed, Wm0, bm0r)

    sccnt = _make_sc_aggregate(npad, h, nch, ch, False)
    cnt = sccnt(x, src3, dst3)
    scagg = _make_sc_aggregate(npad, h, nch, ch, True)
    p0 = scagg(xt0, src3, dst3)

    h1, xt1 = pl.pallas_call(
        functools.partial(_k_update0, d=d, cdim=cdim, g=g),
        grid=(grid,),
        in_specs=[rowblk(d), rowblk(1), full((g, cdim)), pblk(h), pblk(h),
                  full((h, d + cdim + h)), full((1, h)),
                  full((h, h)), full((1, h))],
        out_specs=[rowblk(h), rowblk(h)],
        out_shape=[jax.ShapeDtypeStruct((n, h), F32),
                   jax.ShapeDtypeStruct((n, h), F32)],
    )(x, b2d, substring_embed, p0, cnt, Wu0, bu0r, Wm1, bm1r)

    p1 = scagg(xt1, src3, dst3)

    out2d = pl.pallas_call(
        functools.partial(_k_update1, hdim=h),
        grid=(grid,),
        in_specs=[rowblk(h), pblk(h), pblk(h),
                  full((h, 2 * h)), full((1, h)),
                  full((1, h)), full((1, 1))],
        out_specs=rowblk(1),
        out_shape=jax.ShapeDtypeStruct((n, 1), F32),
    )(h1, p1, cnt, Wu1, bu1r, Wo, bor)

    return out2d.reshape(n)


# speedup vs baseline: 8.4677x; 1.0121x over previous
import functools

import jax
import jax.numpy as jnp
from jax import lax
from jax.experimental import pallas as pl
from jax.experimental.pallas import tpu as pltpu
from jax.experimental.pallas import tpu_sc as plsc

NSC = 2
NSUB = 16
F32 = jnp.float32


def _dot_nt(a, b):
    return lax.dot_general(a, b, (((1,), (1,)), ((), ())),
                           preferred_element_type=F32)


def _largest_div(n, cap):
    for v in range(cap, 0, -1):
        if n % v == 0:
            return v
    return 1


def _make_sc_aggregate(n, h, nch, c, with_gather):
    rpt = n // NSUB

    mesh = plsc.VectorSubcoreMesh(core_axis_name="c", subcore_axis_name="s")
    out_type = jax.ShapeDtypeStruct((NSC, n, h), F32)
    scratch = [
        pltpu.VMEM((2, c), jnp.int32),
        pltpu.VMEM((4, c), jnp.int32),
        pltpu.VMEM((2, c, h), F32),
        pltpu.VMEM_SHARED((n, h), F32),
        pltpu.SemaphoreType.DMA,
        pltpu.SemaphoreType.DMA,
        pltpu.SemaphoreType.DMA,
        pltpu.SemaphoreType.DMA,
        pltpu.SemaphoreType.DMA,
        pltpu.SemaphoreType.DMA,
        pltpu.SemaphoreType.DMA,
        pltpu.SemaphoreType.DMA,
        pltpu.SemaphoreType.DMA,
        pltpu.SemaphoreType.DMA,
    ]

    @functools.partial(pl.kernel, mesh=mesh, out_type=out_type,
                       scratch_types=scratch)
    def sc_kernel(xt_hbm, src_hbm, dst_hbm, p_hbm, sidx, didx, rows,
                  acc_sh, g0, g1, s0, s1, i0, i1, d0, d1, d2, d3):
        cc = lax.axis_index("c")
        ss = lax.axis_index("s")
        tid = cc * NSUB + ss
        gsem = (g0, g1)
        ssem = (s0, s1)
        isem = (i0, i1)
        dsem = (d0, d1, d2, d3)
        zeros16 = jnp.zeros((16,), F32)

        def zrow(j, carry):
            for l in range(h // 16):
                rows[0, j, pl.ds(l * 16, 16)] = zeros16
                rows[1, j, pl.ds(l * 16, 16)] = zeros16
            return carry
        lax.fori_loop(0, c, zrow, 0)

        row0 = ss * rpt
        for k in range(rpt // c):
            pltpu.sync_copy(rows.at[0], acc_sh.at[pl.ds(row0 + k * c, c)])
        if not with_gather:
            ones16 = jnp.ones((16,), F32)

            def orow(j, carry):
                for l in range(h // 16):
                    rows[0, j, pl.ds(l * 16, 16)] = ones16
                    rows[1, j, pl.ds(l * 16, 16)] = ones16
                return carry
            lax.fori_loop(0, c, orow, 0)
        plsc.subcore_barrier()

        ebase = tid * (nch * c)

        def sidx_start(j, p):
            pltpu.async_copy(src_hbm.at[pl.ds(ebase + j * c, c)],
                             sidx.at[p], isem[p])

        def sidx_wait(j, p):
            pltpu.make_async_copy(src_hbm.at[pl.ds(ebase + j * c, c)],
                                  sidx.at[p], isem[p]).wait()

        def didx_start(j, dp):
            pltpu.async_copy(dst_hbm.at[pl.ds(ebase + j * c, c)],
                             didx.at[dp], dsem[dp])

        def didx_wait(j, dp):
            pltpu.make_async_copy(dst_hbm.at[pl.ds(ebase + j * c, c)],
                                  didx.at[dp], dsem[dp]).wait()

        def gather_start(p):
            pltpu.async_copy(xt_hbm.at[sidx.at[p]], rows.at[p], gsem[p])

        def gather_wait(p):
            pltpu.make_async_copy(xt_hbm.at[sidx.at[p]], rows.at[p],
                                  gsem[p]).wait()

        def scatter_start(p, dp):
            pltpu.async_copy(rows.at[p], acc_sh.at[didx.at[dp]], ssem[p],
                             add=True)

        def scatter_wait(p, dp):
            pltpu.make_async_copy(rows.at[p], acc_sh.at[didx.at[dp]],
                                  ssem[p]).wait()

        def step(j, p, dp):
            if with_gather:
                gather_wait(p)
            didx_wait(j, dp)
            scatter_start(p, dp)

            @pl.when(j > 0)
            def _():
                scatter_wait(1 - p, (dp + 3) % 4)

            if with_gather:
                @pl.when(j + 1 < nch)
                def _():
                    sidx_wait(j + 1, 1 - p)
                    gather_start(1 - p)

                @pl.when(j + 2 < nch)
                def _():
                    sidx_start(j + 2, p)

            @pl.when(j + 3 < nch)
            def _():
                didx_start(j + 3, (dp + 3) % 4)

        for j0 in range(min(3, nch)):
            didx_start(j0, j0)
        if with_gather:
            sidx_start(0, 0)
            sidx_wait(0, 0)
            gather_start(0)
            if nch > 1:
                sidx_start(1, 1)

        def quad(i, carry):
            for u in range(4):
                step(4 * i + u, u % 2, u)
            return carry
        lax.fori_loop(0, nch // 4, quad, 0)
        for j in range(nch - nch % 4, nch):
            step(j, j % 2, j % 4)
        scatter_wait((nch - 1) % 2, (nch - 1) % 4)

        plsc.subcore_barrier()
        pltpu.sync_copy(acc_sh.at[pl.ds(row0, rpt)],
                        p_hbm.at[cc, pl.ds(row0, rpt)])

    return sc_kernel


def _onehot(b_blk, g):
    io = lax.broadcasted_iota(jnp.int32, (1, g), 1)
    return (b_blk == io).astype(F32)


def _k_msg0(x_ref, b_ref, se_ref, wm_ref, bm_ref, o_ref, *, d, g):
    wx = wm_ref[:, :d]
    wc = wm_ref[:, d:]
    cw = _dot_nt(se_ref[...], wc)
    oh = _onehot(b_ref[...], g)
    o_ref[...] = (_dot_nt(x_ref[...], wx)
                  + lax.dot_general(oh, cw, (((1,), (0,)), ((), ())),
                                    preferred_element_type=F32)
                  + bm_ref[...])


def _k_update0(x_ref, b_ref, se_ref, p_ref, c_ref, wu_ref, bu_ref,
               wm1_ref, bm1_ref, h1_ref, xt1_ref, *, d, cdim, g):
    pa = p_ref[0] + p_ref[1]
    cnt = c_ref[0, :, 0:1] + c_ref[1, :, 0:1]
    aggr = pa / jnp.maximum(cnt, 1.0)
    wua = wu_ref[:, :d]
    wub = wu_ref[:, d:d + cdim]
    wuc = wu_ref[:, d + cdim:]
    cw = _dot_nt(se_ref[...], wub)
    oh = _onehot(b_ref[...], g)
    h1 = (_dot_nt(x_ref[...], wua)
          + lax.dot_general(oh, cw, (((1,), (0,)), ((), ())),
                            preferred_element_type=F32)
          + _dot_nt(aggr, wuc) + bu_ref[...])
    h1 = jnp.maximum(h1, 0.0)
    h1_ref[...] = h1
    xt1_ref[...] = _dot_nt(h1, wm1_ref[...]) + bm1_ref[...]


def _k_update1(h1_ref, p_ref, c_ref, wu_ref, bu_ref, wo_ref, bo_ref,
               o_ref, *, hdim):
    pa = p_ref[0] + p_ref[1]
    cnt = c_ref[0, :, 0:1] + c_ref[1, :, 0:1]
    aggr = pa / jnp.maximum(cnt, 1.0)
    wa = wu_ref[:, :hdim]
    wb = wu_ref[:, hdim:]
    h2 = _dot_nt(h1_ref[...], wa) + _dot_nt(aggr, wb) + bu_ref[...]
    h2 = jnp.maximum(h2, 0.0)
    o_ref[...] = jnp.sum(h2 * wo_ref[...], axis=1, keepdims=True) + bo_ref[0, 0]


def kernel(x, edge_index, substring_embed, batch, Wm0, bm0, Wu0, bu0,
           Wm1, bm1, Wu1, bu1, Wo, bo):
    n, d = x.shape
    g, cdim = substring_embed.shape
    h = Wm0.shape[0]
    e = edge_index.shape[1]

    ntile = NSC * NSUB
    ept = e // ntile
    ch = _largest_div(ept, 80)
    nch = ept // ch
    npad = -(-n // (8 * NSUB)) * (8 * NSUB)

    bn = _largest_div(n, 2000)
    grid = n // bn

    src3 = edge_index[0]
    dst3 = edge_index[1]
    b2d = batch.reshape(n, 1)
    bm0r = bm0.reshape(1, h)
    bu0r = bu0.reshape(1, h)
    bm1r = bm1.reshape(1, h)
    bu1r = bu1.reshape(1, h)
    bor = bo.reshape(1, 1)

    full = lambda s: pl.BlockSpec(s, lambda i: tuple(0 for _ in s))
    rowblk = lambda w: pl.BlockSpec((bn, w), lambda i: (i, 0))
    pblk = lambda w: pl.BlockSpec((NSC, bn, w), lambda i: (0, i, 0))

    xt0 = pl.pallas_call(
        functools.partial(_k_msg0, d=d, g=g),
        grid=(grid,),
        in_specs=[rowblk(d), rowblk(1), full((g, cdim)),
                  full((h, d + cdim)), full((1, h))],
        out_specs=rowblk(h),
        out_shape=jax.ShapeDtypeStruct((n, h), F32),
    )(x, b2d, substring_embed, Wm0, bm0r)

    sccnt = _make_sc_aggregate(npad, h, nch, ch, False)
    cnt = sccnt(x, src3, dst3)
    scagg = _make_sc_aggregate(npad, h, nch, ch, True)
    p0 = scagg(xt0, src3, dst3)

    h1, xt1 = pl.pallas_call(
        functools.partial(_k_update0, d=d, cdim=cdim, g=g),
        grid=(grid,),
        in_specs=[rowblk(d), rowblk(1), full((g, cdim)), pblk(h), pblk(h),
                  full((h, d + cdim + h)), full((1, h)),
                  full((h, h)), full((1, h))],
        out_specs=[rowblk(h), rowblk(h)],
        out_shape=[jax.ShapeDtypeStruct((n, h), F32),
                   jax.ShapeDtypeStruct((n, h), F32)],
    )(x, b2d, substring_embed, p0, cnt, Wu0, bu0r, Wm1, bm1r)

    p1 = scagg(xt1, src3, dst3)

    out2d = pl.pallas_call(
        functools.partial(_k_update1, hdim=h),
        grid=(grid,),
        in_specs=[rowblk(h), pblk(h), pblk(h),
                  full((h, 2 * h)), full((1, h)),
                  full((1, h)), full((1, 1))],
        out_specs=rowblk(1),
        out_shape=jax.ShapeDtypeStruct((n, 1), F32),
    )(h1, p1, cnt, Wu1, bu1r, Wo, bor)

    return out2d.reshape(n)
